# Initial kernel scaffold; baseline (speedup 1.0000x reference)
import jax
import jax.numpy as jnp
from jax.experimental import pallas as pl


def kernel(x, edge_index, W_pre, b_pre, W_pre2, b_pre2, Wl1, bl1, Wr1, Wl2, bl2, Wr2):
    raise NotImplementedError("write your pallas kernel here")


# speedup vs baseline: 5.5375x; 5.5375x over previous
import functools

import jax
import jax.numpy as jnp
from jax import lax
from jax.experimental import pallas as pl
from jax.experimental.pallas import tpu as pltpu
from jax.experimental.pallas import tpu_sc as plsc

N = 10000
E = 320000
D = 128

NC = 2
NS = 16
NW = NC * NS
EW = E // NW
C = 125
K = EW // C
G = 4
KG = K // G
_STAGE = 3
NPAD = 10240
RPT = NPAD // NS
DW = 16


def _sc_agg_body(h_hbm, src_hbm, dst_hbm, zacc_hbm, part_hbm,
                 acc_sp, sidx_c, didx_c, rows_v, sem):
    cid = lax.axis_index("c")
    sid = lax.axis_index("s")
    wid = sid * NC + cid

    pltpu.sync_copy(zacc_hbm, acc_sp.at[pl.ds(sid * RPT, RPT)])

    plsc.subcore_barrier()

    def chunk(k, _):
        e = wid * K + k
        pltpu.sync_copy(src_hbm.at[e, 0], sidx_c)
        pltpu.sync_copy(dst_hbm.at[e, 0], didx_c)
        pltpu.async_copy(h_hbm.at[sidx_c], rows_v, sem).wait()
        pltpu.sync_copy(rows_v, acc_sp.at[didx_c], add=True)
        return _
    lax.fori_loop(0, K, chunk, None)

    plsc.subcore_barrier()

    pltpu.sync_copy(acc_sp.at[pl.ds(sid * RPT, RPT)],
                    part_hbm.at[cid, pl.ds(sid * RPT, RPT)])


def _make_sc_agg():
    mesh = plsc.VectorSubcoreMesh(core_axis_name="c", subcore_axis_name="s")
    return pl.kernel(
        _sc_agg_body,
        out_type=jax.ShapeDtypeStruct((NC, NPAD, D), jnp.float32),
        mesh=mesh,
        scratch_types=[
            pltpu.VMEM_SHARED((NPAD, D), jnp.float32),
            pltpu.VMEM((C,), jnp.int32),
            pltpu.VMEM((C,), jnp.int32),
            pltpu.VMEM((C, D), jnp.float32),
            pltpu.SemaphoreType.DMA,
        ])


def _sc_deg_body(dst_hbm, zacc_hbm, degp_hbm,
                 deg_sp, didx_c, ones_v):
    cid = lax.axis_index("c")
    sid = lax.axis_index("s")
    wid = sid * NC + cid

    pltpu.sync_copy(zacc_hbm, deg_sp.at[pl.ds(sid * RPT, RPT)])

    ones16 = jnp.full((16,), 1.0, dtype=jnp.float32)

    def init_ones(i, _):
        for j in range(D // 16):
            ones_v[i, pl.ds(j * 16, 16)] = ones16
        return _
    lax.fori_loop(0, C, init_ones, None)

    plsc.subcore_barrier()

    def chunk(k, _):
        pltpu.sync_copy(dst_hbm.at[wid * K + k, 0], didx_c)
        pltpu.sync_copy(ones_v, deg_sp.at[didx_c], add=True)
        return _
    lax.fori_loop(0, K, chunk, None)

    plsc.subcore_barrier()

    pltpu.sync_copy(deg_sp.at[pl.ds(sid * RPT, RPT)],
                    degp_hbm.at[cid, pl.ds(sid * RPT, RPT)])


def _make_sc_deg():
    mesh = plsc.VectorSubcoreMesh(core_axis_name="c", subcore_axis_name="s")
    return pl.kernel(
        _sc_deg_body,
        out_type=jax.ShapeDtypeStruct((NC, NPAD, D), jnp.float32),
        mesh=mesh,
        scratch_types=[
            pltpu.VMEM_SHARED((NPAD, D), jnp.float32),
            pltpu.VMEM((C,), jnp.int32),
            pltpu.VMEM((C, D), jnp.float32),
        ])


_sc_agg = _make_sc_agg()
_sc_deg = _make_sc_deg()


BN = 2000


def _pre_body(x_ref, wp_ref, bp_ref, wp2_ref, bp2_ref, o_ref):
    h = lax.dot_general(x_ref[...], wp_ref[...],
                        (((1,), (1,)), ((), ())),
                        preferred_element_type=jnp.float32)
    h = h + bp_ref[0:1, :]
    h = lax.dot_general(h, wp2_ref[...],
                        (((1,), (1,)), ((), ())),
                        preferred_element_type=jnp.float32)
    o_ref[...] = h + bp2_ref[0:1, :]


def _pre_transform(x, Wp, bp, Wp2, bp2):
    grid = (N // BN,)
    row_spec = pl.BlockSpec((BN, D), lambda i: (i, 0))
    full = pl.BlockSpec((D, D), lambda i: (0, 0))
    bias = pl.BlockSpec((8, D), lambda i: (0, 0))
    return pl.pallas_call(
        _pre_body,
        grid=grid,
        in_specs=[row_spec, full, bias, full, bias],
        out_specs=row_spec,
        out_shape=jax.ShapeDtypeStruct((N, D), jnp.float32),
    )(x, Wp, jnp.broadcast_to(bp, (8, D)), Wp2, jnp.broadcast_to(bp2, (8, D)))


def _combine_body(act, p0_ref, p1_ref, d0_ref, d1_ref, h_ref,
                  wl_ref, bl_ref, wr_ref, o_ref):
    agg = p0_ref[...] + p1_ref[...]
    deg = d0_ref[:, 0:1] + d1_ref[:, 0:1]
    mean = agg / jnp.maximum(deg, 1.0)
    out = lax.dot_general(mean, wl_ref[...], (((1,), (1,)), ((), ())),
                          preferred_element_type=jnp.float32)
    out = out + bl_ref[0:1, :]
    out = out + lax.dot_general(h_ref[...], wr_ref[...],
                                (((1,), (1,)), ((), ())),
                                preferred_element_type=jnp.float32)
    if act == "relu":
        out = jnp.maximum(out, 0.0)
    elif act == "l2norm":
        nrm = jnp.sqrt(jnp.sum(out * out, axis=1, keepdims=True))
        out = out / jnp.maximum(nrm, 1e-12)
    o_ref[...] = out


def _combine(act, p0, p1, d0, d1, h, Wl, bl, Wr):
    grid = (N // BN,)
    row_spec = pl.BlockSpec((BN, D), lambda i: (i, 0))
    full = pl.BlockSpec((D, D), lambda i: (0, 0))
    bias = pl.BlockSpec((8, D), lambda i: (0, 0))
    return pl.pallas_call(
        functools.partial(_combine_body, act),
        grid=grid,
        in_specs=[row_spec, row_spec, row_spec, row_spec, row_spec,
                  full, bias, full],
        out_specs=row_spec,
        out_shape=jax.ShapeDtypeStruct((N, D), jnp.float32),
    )(p0, p1, d0, d1, h, Wl, jnp.broadcast_to(bl, (8, D)), Wr)


def kernel(x, edge_index, W_pre, b_pre, W_pre2, b_pre2,
           Wl1, bl1, Wr1, Wl2, bl2, Wr2):
    src = edge_index[0].reshape(NW * K, 1, C)
    dst = edge_index[1].reshape(NW * K, 1, C)
    zacc = jnp.zeros((RPT, D), jnp.float32)

    h = _pre_transform(x, W_pre, b_pre, W_pre2, b_pre2)
    degp = _sc_deg(dst, zacc)

    part1 = _sc_agg(h, src, dst, zacc)
    h1 = _combine("relu",
                  part1[0, :N], part1[1, :N],
                  degp[0, :N], degp[1, :N],
                  h, Wl1, bl1, Wr1)

    part2 = _sc_agg(h1, src, dst, zacc)
    out = _combine("l2norm",
                   part2[0, :N], part2[1, :N],
                   degp[0, :N], degp[1, :N],
                   h1, Wl2, bl2, Wr2)
    return out


# speedup vs baseline: 7.5819x; 1.3692x over previous
import functools

import jax
import jax.numpy as jnp
from jax import lax
from jax.experimental import pallas as pl
from jax.experimental.pallas import tpu as pltpu
from jax.experimental.pallas import tpu_sc as plsc

N = 10000
E = 320000
D = 128

NC = 2
NS = 16
NW = NC * NS
EW = E // NW
C = 125
K = EW // C
G = 4
KG = K // G
_STAGE = 3
NPAD = 10240
RPT = NPAD // NS
DW = 16


def _sc_agg_body(h_hbm, src_hbm, dst_hbm, zacc_hbm, part_hbm,
                 acc_sp, sidx_g, didx_g, rows_a, rows_b,
                 sem_ga, sem_gb, sem_sa, sem_sb):
    cid = lax.axis_index("c")
    sid = lax.axis_index("s")
    wid = sid * NC + cid

    pltpu.sync_copy(zacc_hbm, acc_sp.at[pl.ds(sid * RPT, RPT)])

    plsc.subcore_barrier()

    def group(g, _):
        base = wid * G + g
        pltpu.sync_copy(src_hbm.at[base], sidx_g)
        pltpu.sync_copy(dst_hbm.at[base], didx_g)

        def pair(j, _):
            ga = pltpu.async_copy(h_hbm.at[sidx_g.at[2 * j, 0]],
                                  rows_a, sem_ga)
            gb = pltpu.async_copy(h_hbm.at[sidx_g.at[2 * j + 1, 0]],
                                  rows_b, sem_gb)
            ga.wait()
            sa = pltpu.async_copy(rows_a, acc_sp.at[didx_g.at[2 * j, 0]],
                                  sem_sa, add=True)
            gb.wait()
            sb = pltpu.async_copy(rows_b,
                                  acc_sp.at[didx_g.at[2 * j + 1, 0]],
                                  sem_sb, add=True)
            sa.wait()
            sb.wait()
            return _
        lax.fori_loop(0, KG // 2, pair, None)
        return _
    lax.fori_loop(0, G, group, None)

    plsc.subcore_barrier()

    pltpu.sync_copy(acc_sp.at[pl.ds(sid * RPT, RPT)],
                    part_hbm.at[cid, pl.ds(sid * RPT, RPT)])


def _make_sc_agg():
    mesh = plsc.VectorSubcoreMesh(core_axis_name="c", subcore_axis_name="s")
    return pl.kernel(
        _sc_agg_body,
        out_type=jax.ShapeDtypeStruct((NC, NPAD, D), jnp.float32),
        mesh=mesh,
        scratch_types=[
            pltpu.VMEM_SHARED((NPAD, D), jnp.float32),
            pltpu.VMEM((KG, 1, C), jnp.int32),
            pltpu.VMEM((KG, 1, C), jnp.int32),
            pltpu.VMEM((C, D), jnp.float32),
            pltpu.VMEM((C, D), jnp.float32),
            pltpu.SemaphoreType.DMA,
            pltpu.SemaphoreType.DMA,
            pltpu.SemaphoreType.DMA,
            pltpu.SemaphoreType.DMA,
        ])


def _sc_deg_body(dst_hbm, zacc_hbm, degp_hbm,
                 deg_sp, didx_c, ones_v):
    cid = lax.axis_index("c")
    sid = lax.axis_index("s")
    wid = sid * NC + cid

    pltpu.sync_copy(zacc_hbm, deg_sp.at[pl.ds(sid * RPT, RPT)])

    ones16 = jnp.full((16,), 1.0, dtype=jnp.float32)

    def init_ones(i, _):
        for j in range(D // 16):
            ones_v[i, pl.ds(j * 16, 16)] = ones16
        return _
    lax.fori_loop(0, C, init_ones, None)

    plsc.subcore_barrier()

    def chunk(k, _):
        pltpu.sync_copy(dst_hbm.at[wid * K + k, 0], didx_c)
        pltpu.sync_copy(ones_v, deg_sp.at[didx_c], add=True)
        return _
    lax.fori_loop(0, K, chunk, None)

    plsc.subcore_barrier()

    pltpu.sync_copy(deg_sp.at[pl.ds(sid * RPT, RPT)],
                    degp_hbm.at[cid, pl.ds(sid * RPT, RPT)])


def _make_sc_deg():
    mesh = plsc.VectorSubcoreMesh(core_axis_name="c", subcore_axis_name="s")
    return pl.kernel(
        _sc_deg_body,
        out_type=jax.ShapeDtypeStruct((NC, NPAD, D), jnp.float32),
        mesh=mesh,
        scratch_types=[
            pltpu.VMEM_SHARED((NPAD, D), jnp.float32),
            pltpu.VMEM((C,), jnp.int32),
            pltpu.VMEM((C, D), jnp.float32),
        ])


_sc_agg = _make_sc_agg()
_sc_deg = _make_sc_deg()


BN = 2000


def _pre_body(x_ref, wp_ref, bp_ref, wp2_ref, bp2_ref, o_ref):
    h = lax.dot_general(x_ref[...], wp_ref[...],
                        (((1,), (1,)), ((), ())),
                        preferred_element_type=jnp.float32)
    h = h + bp_ref[0:1, :]
    h = lax.dot_general(h, wp2_ref[...],
                        (((1,), (1,)), ((), ())),
                        preferred_element_type=jnp.float32)
    o_ref[...] = h + bp2_ref[0:1, :]


def _pre_transform(x, Wp, bp, Wp2, bp2):
    grid = (N // BN,)
    row_spec = pl.BlockSpec((BN, D), lambda i: (i, 0))
    full = pl.BlockSpec((D, D), lambda i: (0, 0))
    bias = pl.BlockSpec((8, D), lambda i: (0, 0))
    return pl.pallas_call(
        _pre_body,
        grid=grid,
        in_specs=[row_spec, full, bias, full, bias],
        out_specs=row_spec,
        out_shape=jax.ShapeDtypeStruct((N, D), jnp.float32),
    )(x, Wp, jnp.broadcast_to(bp, (8, D)), Wp2, jnp.broadcast_to(bp2, (8, D)))


def _combine_body(act, p0_ref, p1_ref, d0_ref, d1_ref, h_ref,
                  wl_ref, bl_ref, wr_ref, o_ref):
    agg = p0_ref[...] + p1_ref[...]
    deg = d0_ref[:, 0:1] + d1_ref[:, 0:1]
    mean = agg / jnp.maximum(deg, 1.0)
    out = lax.dot_general(mean, wl_ref[...], (((1,), (1,)), ((), ())),
                          preferred_element_type=jnp.float32)
    out = out + bl_ref[0:1, :]
    out = out + lax.dot_general(h_ref[...], wr_ref[...],
                                (((1,), (1,)), ((), ())),
                                preferred_element_type=jnp.float32)
    if act == "relu":
        out = jnp.maximum(out, 0.0)
    elif act == "l2norm":
        nrm = jnp.sqrt(jnp.sum(out * out, axis=1, keepdims=True))
        out = out / jnp.maximum(nrm, 1e-12)
    o_ref[...] = out


def _combine(act, p0, p1, d0, d1, h, Wl, bl, Wr):
    grid = (N // BN,)
    row_spec = pl.BlockSpec((BN, D), lambda i: (i, 0))
    full = pl.BlockSpec((D, D), lambda i: (0, 0))
    bias = pl.BlockSpec((8, D), lambda i: (0, 0))
    return pl.pallas_call(
        functools.partial(_combine_body, act),
        grid=grid,
        in_specs=[row_spec, row_spec, row_spec, row_spec, row_spec,
                  full, bias, full],
        out_specs=row_spec,
        out_shape=jax.ShapeDtypeStruct((N, D), jnp.float32),
    )(p0, p1, d0, d1, h, Wl, jnp.broadcast_to(bl, (8, D)), Wr)


def kernel(x, edge_index, W_pre, b_pre, W_pre2, b_pre2,
           Wl1, bl1, Wr1, Wl2, bl2, Wr2):
    src = edge_index[0].reshape(NW * G, KG, 1, C)
    dst = edge_index[1].reshape(NW * G, KG, 1, C)
    dstc = edge_index[1].reshape(NW * K, 1, C)
    zacc = jnp.zeros((RPT, D), jnp.float32)

    h = _pre_transform(x, W_pre, b_pre, W_pre2, b_pre2)
    degp = _sc_deg(dstc, zacc)

    part1 = _sc_agg(h, src, dst, zacc)
    h1 = _combine("relu",
                  part1[0, :N], part1[1, :N],
                  degp[0, :N], degp[1, :N],
                  h, Wl1, bl1, Wr1)

    part2 = _sc_agg(h1, src, dst, zacc)
    out = _combine("l2norm",
                   part2[0, :N], part2[1, :N],
                   degp[0, :N], degp[1, :N],
                   h1, Wl2, bl2, Wr2)
    return out


# speedup vs baseline: 8.1031x; 1.0687x over previous
import functools

import jax
import jax.numpy as jnp
from jax import lax
from jax.experimental import pallas as pl
from jax.experimental.pallas import tpu as pltpu
from jax.experimental.pallas import tpu_sc as plsc

N = 10000
E = 320000
D = 128

NC = 2
NS = 16
NW = NC * NS
EW = E // NW
C = 125
K = EW // C
G = 4
KG = K // G
_STAGE = 3
NPAD = 10240
RPT = NPAD // NS
DW = 16


def _sc_agg_body(h_hbm, src_hbm, dst_hbm, zacc_hbm, part_hbm,
                 acc_sp, sidx_g, didx_g, rows_a, rows_b,
                 sem_ga, sem_gb, sem_sa, sem_sb):
    cid = lax.axis_index("c")
    sid = lax.axis_index("s")
    wid = sid * NC + cid

    pltpu.sync_copy(zacc_hbm, acc_sp.at[pl.ds(sid * RPT, RPT)])

    plsc.subcore_barrier()

    def group(g, _):
        base = wid * G + g
        pltpu.sync_copy(src_hbm.at[base], sidx_g)
        pltpu.sync_copy(dst_hbm.at[base], didx_g)

        def pair(j, _):
            ga = pltpu.async_copy(h_hbm.at[sidx_g.at[2 * j, 0]],
                                  rows_a, sem_ga)
            gb = pltpu.async_copy(h_hbm.at[sidx_g.at[2 * j + 1, 0]],
                                  rows_b, sem_gb)
            ga.wait()
            sa = pltpu.async_copy(rows_a, acc_sp.at[didx_g.at[2 * j, 0]],
                                  sem_sa, add=True)
            gb.wait()
            sb = pltpu.async_copy(rows_b,
                                  acc_sp.at[didx_g.at[2 * j + 1, 0]],
                                  sem_sb, add=True)
            sa.wait()
            sb.wait()
            return _
        lax.fori_loop(0, KG // 2, pair, None)
        return _
    lax.fori_loop(0, G, group, None)

    plsc.subcore_barrier()

    pltpu.sync_copy(acc_sp.at[pl.ds(sid * RPT, RPT)],
                    part_hbm.at[cid, pl.ds(sid * RPT, RPT)])


def _make_sc_agg():
    mesh = plsc.VectorSubcoreMesh(core_axis_name="c", subcore_axis_name="s")
    return pl.kernel(
        _sc_agg_body,
        out_type=jax.ShapeDtypeStruct((NC, NPAD, D), jnp.float32),
        mesh=mesh,
        scratch_types=[
            pltpu.VMEM_SHARED((NPAD, D), jnp.float32),
            pltpu.VMEM((KG, 1, C), jnp.int32),
            pltpu.VMEM((KG, 1, C), jnp.int32),
            pltpu.VMEM((C, D), jnp.float32),
            pltpu.VMEM((C, D), jnp.float32),
            pltpu.SemaphoreType.DMA,
            pltpu.SemaphoreType.DMA,
            pltpu.SemaphoreType.DMA,
            pltpu.SemaphoreType.DMA,
        ])


def _sc_deg_body(dst_hbm, zacc_hbm, degp_hbm,
                 deg_sp, didx_g, ones_v, sem_s):
    cid = lax.axis_index("c")
    sid = lax.axis_index("s")
    wid = sid * NC + cid

    pltpu.sync_copy(zacc_hbm, deg_sp.at[pl.ds(sid * RPT, RPT)])

    ones16 = jnp.full((16,), 1.0, dtype=jnp.float32)

    def init_ones(i, _):
        for j in range(D // 16):
            ones_v[i, pl.ds(j * 16, 16)] = ones16
        return _
    lax.fori_loop(0, C, init_ones, None)

    plsc.subcore_barrier()

    def group(g, _):
        pltpu.sync_copy(dst_hbm.at[wid * G + g], didx_g)

        def fire(j, _):
            pltpu.async_copy(ones_v, deg_sp.at[didx_g.at[j, 0]],
                             sem_s, add=True)
            return _
        lax.fori_loop(0, KG, fire, None)

        def drain(j, _):
            pltpu.make_async_copy(ones_v, deg_sp.at[didx_g.at[0, 0]],
                                  sem_s).wait()
            return _
        lax.fori_loop(0, KG, drain, None)
        return _
    lax.fori_loop(0, G, group, None)

    plsc.subcore_barrier()

    pltpu.sync_copy(deg_sp.at[pl.ds(sid * RPT, RPT)],
                    degp_hbm.at[cid, pl.ds(sid * RPT, RPT)])


def _make_sc_deg():
    mesh = plsc.VectorSubcoreMesh(core_axis_name="c", subcore_axis_name="s")
    return pl.kernel(
        _sc_deg_body,
        out_type=jax.ShapeDtypeStruct((NC, NPAD, D), jnp.float32),
        mesh=mesh,
        scratch_types=[
            pltpu.VMEM_SHARED((NPAD, D), jnp.float32),
            pltpu.VMEM((KG, 1, C), jnp.int32),
            pltpu.VMEM((C, D), jnp.float32),
            pltpu.SemaphoreType.DMA,
        ])


_sc_agg = _make_sc_agg()
_sc_deg = _make_sc_deg()


BN = 2000


def _pre_body(x_ref, wp_ref, bp_ref, wp2_ref, bp2_ref, o_ref):
    h = lax.dot_general(x_ref[...], wp_ref[...],
                        (((1,), (1,)), ((), ())),
                        preferred_element_type=jnp.float32)
    h = h + bp_ref[0:1, :]
    h = lax.dot_general(h, wp2_ref[...],
                        (((1,), (1,)), ((), ())),
                        preferred_element_type=jnp.float32)
    o_ref[...] = h + bp2_ref[0:1, :]


def _pre_transform(x, Wp, bp, Wp2, bp2):
    grid = (N // BN,)
    row_spec = pl.BlockSpec((BN, D), lambda i: (i, 0))
    full = pl.BlockSpec((D, D), lambda i: (0, 0))
    bias = pl.BlockSpec((8, D), lambda i: (0, 0))
    return pl.pallas_call(
        _pre_body,
        grid=grid,
        in_specs=[row_spec, full, bias, full, bias],
        out_specs=row_spec,
        out_shape=jax.ShapeDtypeStruct((N, D), jnp.float32),
    )(x, Wp, jnp.broadcast_to(bp, (8, D)), Wp2, jnp.broadcast_to(bp2, (8, D)))


def _combine_body(act, p0_ref, p1_ref, d0_ref, d1_ref, h_ref,
                  wl_ref, bl_ref, wr_ref, o_ref):
    agg = p0_ref[...] + p1_ref[...]
    deg = d0_ref[:, 0:1] + d1_ref[:, 0:1]
    mean = agg / jnp.maximum(deg, 1.0)
    out = lax.dot_general(mean, wl_ref[...], (((1,), (1,)), ((), ())),
                          preferred_element_type=jnp.float32)
    out = out + bl_ref[0:1, :]
    out = out + lax.dot_general(h_ref[...], wr_ref[...],
                                (((1,), (1,)), ((), ())),
                                preferred_element_type=jnp.float32)
    if act == "relu":
        out = jnp.maximum(out, 0.0)
    elif act == "l2norm":
        nrm = jnp.sqrt(jnp.sum(out * out, axis=1, keepdims=True))
        out = out / jnp.maximum(nrm, 1e-12)
    o_ref[...] = out


def _combine(act, p0, p1, d0, d1, h, Wl, bl, Wr):
    grid = (N // BN,)
    row_spec = pl.BlockSpec((BN, D), lambda i: (i, 0))
    full = pl.BlockSpec((D, D), lambda i: (0, 0))
    bias = pl.BlockSpec((8, D), lambda i: (0, 0))
    return pl.pallas_call(
        functools.partial(_combine_body, act),
        grid=grid,
        in_specs=[row_spec, row_spec, row_spec, row_spec, row_spec,
                  full, bias, full],
        out_specs=row_spec,
        out_shape=jax.ShapeDtypeStruct((N, D), jnp.float32),
    )(p0, p1, d0, d1, h, Wl, jnp.broadcast_to(bl, (8, D)), Wr)


def kernel(x, edge_index, W_pre, b_pre, W_pre2, b_pre2,
           Wl1, bl1, Wr1, Wl2, bl2, Wr2):
    src = edge_index[0].reshape(NW * G, KG, 1, C)
    dst = edge_index[1].reshape(NW * G, KG, 1, C)
    zacc = jnp.zeros((RPT, D), jnp.float32)

    h = _pre_transform(x, W_pre, b_pre, W_pre2, b_pre2)
    degp = _sc_deg(dst, zacc)

    part1 = _sc_agg(h, src, dst, zacc)
    h1 = _combine("relu",
                  part1[0, :N], part1[1, :N],
                  degp[0, :N], degp[1, :N],
                  h, Wl1, bl1, Wr1)

    part2 = _sc_agg(h1, src, dst, zacc)
    out = _combine("l2norm",
                   part2[0, :N], part2[1, :N],
                   degp[0, :N], degp[1, :N],
                   h1, Wl2, bl2, Wr2)
    return out


# speedup vs baseline: 8.1918x; 1.0109x over previous
import functools

import jax
import jax.numpy as jnp
from jax import lax
from jax.experimental import pallas as pl
from jax.experimental.pallas import tpu as pltpu
from jax.experimental.pallas import tpu_sc as plsc

N = 10000
E = 320000
D = 128

NC = 2
NS = 16
NW = NC * NS
EW = E // NW
C = 125
K = EW // C
G = 4
KG = K // G
_STAGE = 3
NPAD = 10240
RPT = NPAD // NS
DW = 16


def _sc_agg_body(h_hbm, src_hbm, dst_hbm, zacc_hbm, part_hbm,
                 acc_sp, sidx_g, didx_g, rows_a, rows_b,
                 sem_ga, sem_gb, sem_sa, sem_sb):
    cid = lax.axis_index("c")
    sid = lax.axis_index("s")
    wid = sid * NC + cid

    pltpu.sync_copy(zacc_hbm, acc_sp.at[pl.ds(sid * RPT, RPT)])

    plsc.subcore_barrier()

    def gather_start(k, rows, sem):
        pltpu.async_copy(h_hbm.at[sidx_g.at[k, 0]], rows, sem)

    def gather_wait(k, rows, sem):
        pltpu.make_async_copy(h_hbm.at[sidx_g.at[k, 0]], rows, sem).wait()

    def scat_start(k, rows, sem):
        pltpu.async_copy(rows, acc_sp.at[didx_g.at[k, 0]], sem, add=True)

    def scat_wait(rows, sem):
        pltpu.make_async_copy(rows, acc_sp.at[didx_g.at[0, 0]], sem).wait()

    P = KG // 2

    def group(g, _):
        base = wid * G + g
        pltpu.sync_copy(src_hbm.at[base], sidx_g)
        pltpu.sync_copy(dst_hbm.at[base], didx_g)

        gather_start(0, rows_a, sem_ga)
        gather_start(1, rows_b, sem_gb)

        def pair(j, _):
            a = 2 * j
            gather_wait(a, rows_a, sem_ga)
            scat_start(a, rows_a, sem_sa)
            gather_wait(a + 1, rows_b, sem_gb)
            scat_start(a + 1, rows_b, sem_sb)
            scat_wait(rows_a, sem_sa)
            gather_start(a + 2, rows_a, sem_ga)
            scat_wait(rows_b, sem_sb)
            gather_start(a + 3, rows_b, sem_gb)
            return _
        lax.fori_loop(0, P - 1, pair, None)

        a = 2 * (P - 1)
        gather_wait(a, rows_a, sem_ga)
        scat_start(a, rows_a, sem_sa)
        gather_wait(a + 1, rows_b, sem_gb)
        scat_start(a + 1, rows_b, sem_sb)
        scat_wait(rows_a, sem_sa)
        scat_wait(rows_b, sem_sb)
        return _
    lax.fori_loop(0, G, group, None)

    plsc.subcore_barrier()

    pltpu.sync_copy(acc_sp.at[pl.ds(sid * RPT, RPT)],
                    part_hbm.at[cid, pl.ds(sid * RPT, RPT)])


def _make_sc_agg():
    mesh = plsc.VectorSubcoreMesh(core_axis_name="c", subcore_axis_name="s")
    return pl.kernel(
        _sc_agg_body,
        out_type=jax.ShapeDtypeStruct((NC, NPAD, D), jnp.float32),
        mesh=mesh,
        scratch_types=[
            pltpu.VMEM_SHARED((NPAD, D), jnp.float32),
            pltpu.VMEM((KG, 1, C), jnp.int32),
            pltpu.VMEM((KG, 1, C), jnp.int32),
            pltpu.VMEM((C, D), jnp.float32),
            pltpu.VMEM((C, D), jnp.float32),
            pltpu.SemaphoreType.DMA,
            pltpu.SemaphoreType.DMA,
            pltpu.SemaphoreType.DMA,
            pltpu.SemaphoreType.DMA,
        ])


def _sc_deg_body(dst_hbm, zacc_hbm, degp_hbm,
                 deg_sp, didx_g, ones_v, sem_s):
    cid = lax.axis_index("c")
    sid = lax.axis_index("s")
    wid = sid * NC + cid

    pltpu.sync_copy(zacc_hbm, deg_sp.at[pl.ds(sid * RPT, RPT)])

    ones16 = jnp.full((16,), 1.0, dtype=jnp.float32)

    def init_ones(i, _):
        for j in range(D // 16):
            ones_v[i, pl.ds(j * 16, 16)] = ones16
        return _
    lax.fori_loop(0, C, init_ones, None)

    plsc.subcore_barrier()

    def group(g, _):
        pltpu.sync_copy(dst_hbm.at[wid * G + g], didx_g)

        def fire(j, _):
            pltpu.async_copy(ones_v, deg_sp.at[didx_g.at[j, 0]],
                             sem_s, add=True)
            return _
        lax.fori_loop(0, KG, fire, None)

        def drain(j, _):
            pltpu.make_async_copy(ones_v, deg_sp.at[didx_g.at[0, 0]],
                                  sem_s).wait()
            return _
        lax.fori_loop(0, KG, drain, None)
        return _
    lax.fori_loop(0, G, group, None)

    plsc.subcore_barrier()

    pltpu.sync_copy(deg_sp.at[pl.ds(sid * RPT, RPT)],
                    degp_hbm.at[cid, pl.ds(sid * RPT, RPT)])


def _make_sc_deg():
    mesh = plsc.VectorSubcoreMesh(core_axis_name="c", subcore_axis_name="s")
    return pl.kernel(
        _sc_deg_body,
        out_type=jax.ShapeDtypeStruct((NC, NPAD, D), jnp.float32),
        mesh=mesh,
        scratch_types=[
            pltpu.VMEM_SHARED((NPAD, D), jnp.float32),
            pltpu.VMEM((KG, 1, C), jnp.int32),
            pltpu.VMEM((C, D), jnp.float32),
            pltpu.SemaphoreType.DMA,
        ])


_sc_agg = _make_sc_agg()
_sc_deg = _make_sc_deg()


BN = 2000


def _pre_body(x_ref, wp_ref, bp_ref, wp2_ref, bp2_ref, o_ref):
    h = lax.dot_general(x_ref[...], wp_ref[...],
                        (((1,), (1,)), ((), ())),
                        preferred_element_type=jnp.float32)
    h = h + bp_ref[0:1, :]
    h = lax.dot_general(h, wp2_ref[...],
                        (((1,), (1,)), ((), ())),
                        preferred_element_type=jnp.float32)
    o_ref[...] = h + bp2_ref[0:1, :]


def _pre_transform(x, Wp, bp, Wp2, bp2):
    grid = (N // BN,)
    row_spec = pl.BlockSpec((BN, D), lambda i: (i, 0))
    full = pl.BlockSpec((D, D), lambda i: (0, 0))
    bias = pl.BlockSpec((8, D), lambda i: (0, 0))
    return pl.pallas_call(
        _pre_body,
        grid=grid,
        in_specs=[row_spec, full, bias, full, bias],
        out_specs=row_spec,
        out_shape=jax.ShapeDtypeStruct((N, D), jnp.float32),
    )(x, Wp, jnp.broadcast_to(bp, (8, D)), Wp2, jnp.broadcast_to(bp2, (8, D)))


def _combine_body(act, p0_ref, p1_ref, d0_ref, d1_ref, h_ref,
                  wl_ref, bl_ref, wr_ref, o_ref):
    agg = p0_ref[...] + p1_ref[...]
    deg = d0_ref[:, 0:1] + d1_ref[:, 0:1]
    mean = agg / jnp.maximum(deg, 1.0)
    out = lax.dot_general(mean, wl_ref[...], (((1,), (1,)), ((), ())),
                          preferred_element_type=jnp.float32)
    out = out + bl_ref[0:1, :]
    out = out + lax.dot_general(h_ref[...], wr_ref[...],
                                (((1,), (1,)), ((), ())),
                                preferred_element_type=jnp.float32)
    if act == "relu":
        out = jnp.maximum(out, 0.0)
    elif act == "l2norm":
        nrm = jnp.sqrt(jnp.sum(out * out, axis=1, keepdims=True))
        out = out / jnp.maximum(nrm, 1e-12)
    o_ref[...] = out


def _combine(act, p0, p1, d0, d1, h, Wl, bl, Wr):
    grid = (N // BN,)
    row_spec = pl.BlockSpec((BN, D), lambda i: (i, 0))
    full = pl.BlockSpec((D, D), lambda i: (0, 0))
    bias = pl.BlockSpec((8, D), lambda i: (0, 0))
    return pl.pallas_call(
        functools.partial(_combine_body, act),
        grid=grid,
        in_specs=[row_spec, row_spec, row_spec, row_spec, row_spec,
                  full, bias, full],
        out_specs=row_spec,
        out_shape=jax.ShapeDtypeStruct((N, D), jnp.float32),
    )(p0, p1, d0, d1, h, Wl, jnp.broadcast_to(bl, (8, D)), Wr)


def kernel(x, edge_index, W_pre, b_pre, W_pre2, b_pre2,
           Wl1, bl1, Wr1, Wl2, bl2, Wr2):
    src = edge_index[0].reshape(NW * G, KG, 1, C)
    dst = edge_index[1].reshape(NW * G, KG, 1, C)
    zacc = jnp.zeros((RPT, D), jnp.float32)

    h = _pre_transform(x, W_pre, b_pre, W_pre2, b_pre2)
    degp = _sc_deg(dst, zacc)

    part1 = _sc_agg(h, src, dst, zacc)
    h1 = _combine("relu",
                  part1[0, :N], part1[1, :N],
                  degp[0, :N], degp[1, :N],
                  h, Wl1, bl1, Wr1)

    part2 = _sc_agg(h1, src, dst, zacc)
    out = _combine("l2norm",
                   part2[0, :N], part2[1, :N],
                   degp[0, :N], degp[1, :N],
                   h1, Wl2, bl2, Wr2)
    return out


# speedup vs baseline: 8.3035x; 1.0136x over previous
import functools

import jax
import jax.numpy as jnp
from jax import lax
from jax.experimental import pallas as pl
from jax.experimental.pallas import tpu as pltpu
from jax.experimental.pallas import tpu_sc as plsc

N = 10000
E = 320000
D = 128

NC = 2
NS = 16
NW = NC * NS
EW = E // NW
C = 125
K = EW // C
G = 4
KG = K // G
_STAGE = 3
NPAD = 10240
RPT = NPAD // NS
DW = 16


def _sc_agg_body(with_deg, h_hbm, src_hbm, dst_hbm, zacc_hbm, part_hbm,
                 *rest):
    if with_deg:
        degp_hbm, acc_sp, sidx_g, didx_g, rows_a, rows_b, \
            sem_ga, sem_gb, sem_sa, sem_sb = rest
    else:
        degp_hbm = None
        acc_sp, sidx_g, didx_g, rows_a, rows_b, \
            sem_ga, sem_gb, sem_sa, sem_sb = rest
    cid = lax.axis_index("c")
    sid = lax.axis_index("s")
    wid = sid * NC + cid

    if with_deg:
        pltpu.sync_copy(zacc_hbm, acc_sp.at[pl.ds(sid * RPT, RPT)])

        ones16 = jnp.full((16,), 1.0, dtype=jnp.float32)

        def init_ones(i, _):
            for j in range(D // 16):
                rows_a[i, pl.ds(j * 16, 16)] = ones16
            return _
        lax.fori_loop(0, C, init_ones, None)

        plsc.subcore_barrier()

        def dgroup(g, _):
            pltpu.sync_copy(dst_hbm.at[wid * G + g], didx_g)

            def fire(j, _):
                pltpu.async_copy(rows_a, acc_sp.at[didx_g.at[j, 0]],
                                 sem_sa, add=True)
                return _
            lax.fori_loop(0, KG, fire, None)

            def drain(j, _):
                pltpu.make_async_copy(rows_a, acc_sp.at[didx_g.at[0, 0]],
                                      sem_sa).wait()
                return _
            lax.fori_loop(0, KG, drain, None)
            return _
        lax.fori_loop(0, G, dgroup, None)

        plsc.subcore_barrier()

        pltpu.sync_copy(acc_sp.at[pl.ds(sid * RPT, RPT)],
                        degp_hbm.at[cid, pl.ds(sid * RPT, RPT)])

    pltpu.sync_copy(zacc_hbm, acc_sp.at[pl.ds(sid * RPT, RPT)])

    plsc.subcore_barrier()

    def gather_start(k, rows, sem):
        pltpu.async_copy(h_hbm.at[sidx_g.at[k, 0]], rows, sem)

    def gather_wait(k, rows, sem):
        pltpu.make_async_copy(h_hbm.at[sidx_g.at[k, 0]], rows, sem).wait()

    def scat_start(k, rows, sem):
        pltpu.async_copy(rows, acc_sp.at[didx_g.at[k, 0]], sem, add=True)

    def scat_wait(rows, sem):
        pltpu.make_async_copy(rows, acc_sp.at[didx_g.at[0, 0]], sem).wait()

    P = KG // 2

    def group(g, _):
        base = wid * G + g
        pltpu.sync_copy(src_hbm.at[base], sidx_g)
        pltpu.sync_copy(dst_hbm.at[base], didx_g)

        gather_start(0, rows_a, sem_ga)
        gather_start(1, rows_b, sem_gb)

        def pair(j, _):
            a = 2 * j
            gather_wait(a, rows_a, sem_ga)
            scat_start(a, rows_a, sem_sa)
            gather_wait(a + 1, rows_b, sem_gb)
            scat_start(a + 1, rows_b, sem_sb)
            scat_wait(rows_a, sem_sa)
            gather_start(a + 2, rows_a, sem_ga)
            scat_wait(rows_b, sem_sb)
            gather_start(a + 3, rows_b, sem_gb)
            return _
        lax.fori_loop(0, P - 1, pair, None)

        a = 2 * (P - 1)
        gather_wait(a, rows_a, sem_ga)
        scat_start(a, rows_a, sem_sa)
        gather_wait(a + 1, rows_b, sem_gb)
        scat_start(a + 1, rows_b, sem_sb)
        scat_wait(rows_a, sem_sa)
        scat_wait(rows_b, sem_sb)
        return _
    lax.fori_loop(0, G, group, None)

    plsc.subcore_barrier()

    pltpu.sync_copy(acc_sp.at[pl.ds(sid * RPT, RPT)],
                    part_hbm.at[cid, pl.ds(sid * RPT, RPT)])


def _make_sc_agg(with_deg):
    mesh = plsc.VectorSubcoreMesh(core_axis_name="c", subcore_axis_name="s")
    out_type = [jax.ShapeDtypeStruct((NC, NPAD, D), jnp.float32)]
    if with_deg:
        out_type.append(jax.ShapeDtypeStruct((NC, NPAD, D), jnp.float32))
    return pl.kernel(
        functools.partial(_sc_agg_body, with_deg),
        out_type=tuple(out_type) if with_deg else out_type[0],
        mesh=mesh,
        scratch_types=[
            pltpu.VMEM_SHARED((NPAD, D), jnp.float32),
            pltpu.VMEM((KG, 1, C), jnp.int32),
            pltpu.VMEM((KG, 1, C), jnp.int32),
            pltpu.VMEM((C, D), jnp.float32),
            pltpu.VMEM((C, D), jnp.float32),
            pltpu.SemaphoreType.DMA,
            pltpu.SemaphoreType.DMA,
            pltpu.SemaphoreType.DMA,
            pltpu.SemaphoreType.DMA,
        ])


_sc_agg_deg = _make_sc_agg(True)
_sc_agg = _make_sc_agg(False)


BN = 2000


def _pre_body(x_ref, wp_ref, bp_ref, wp2_ref, bp2_ref, o_ref):
    h = lax.dot_general(x_ref[...], wp_ref[...],
                        (((1,), (1,)), ((), ())),
                        preferred_element_type=jnp.float32)
    h = h + bp_ref[0:1, :]
    h = lax.dot_general(h, wp2_ref[...],
                        (((1,), (1,)), ((), ())),
                        preferred_element_type=jnp.float32)
    o_ref[...] = h + bp2_ref[0:1, :]


def _pre_transform(x, Wp, bp, Wp2, bp2):
    grid = (N // BN,)
    row_spec = pl.BlockSpec((BN, D), lambda i: (i, 0))
    full = pl.BlockSpec((D, D), lambda i: (0, 0))
    bias = pl.BlockSpec((8, D), lambda i: (0, 0))
    return pl.pallas_call(
        _pre_body,
        grid=grid,
        in_specs=[row_spec, full, bias, full, bias],
        out_specs=row_spec,
        out_shape=jax.ShapeDtypeStruct((N, D), jnp.float32),
    )(x, Wp, jnp.broadcast_to(bp, (8, D)), Wp2, jnp.broadcast_to(bp2, (8, D)))


def _combine_body(act, p0_ref, p1_ref, d0_ref, d1_ref, h_ref,
                  wl_ref, bl_ref, wr_ref, o_ref):
    agg = p0_ref[...] + p1_ref[...]
    deg = d0_ref[:, 0:1] + d1_ref[:, 0:1]
    mean = agg / jnp.maximum(deg, 1.0)
    out = lax.dot_general(mean, wl_ref[...], (((1,), (1,)), ((), ())),
                          preferred_element_type=jnp.float32)
    out = out + bl_ref[0:1, :]
    out = out + lax.dot_general(h_ref[...], wr_ref[...],
                                (((1,), (1,)), ((), ())),
                                preferred_element_type=jnp.float32)
    if act == "relu":
        out = jnp.maximum(out, 0.0)
    elif act == "l2norm":
        nrm = jnp.sqrt(jnp.sum(out * out, axis=1, keepdims=True))
        out = out / jnp.maximum(nrm, 1e-12)
    o_ref[...] = out


def _combine(act, p0, p1, d0, d1, h, Wl, bl, Wr):
    grid = (N // BN,)
    row_spec = pl.BlockSpec((BN, D), lambda i: (i, 0))
    full = pl.BlockSpec((D, D), lambda i: (0, 0))
    bias = pl.BlockSpec((8, D), lambda i: (0, 0))
    return pl.pallas_call(
        functools.partial(_combine_body, act),
        grid=grid,
        in_specs=[row_spec, row_spec, row_spec, row_spec, row_spec,
                  full, bias, full],
        out_specs=row_spec,
        out_shape=jax.ShapeDtypeStruct((N, D), jnp.float32),
    )(p0, p1, d0, d1, h, Wl, jnp.broadcast_to(bl, (8, D)), Wr)


def kernel(x, edge_index, W_pre, b_pre, W_pre2, b_pre2,
           Wl1, bl1, Wr1, Wl2, bl2, Wr2):
    src = edge_index[0].reshape(NW * G, KG, 1, C)
    dst = edge_index[1].reshape(NW * G, KG, 1, C)
    zacc = jnp.zeros((RPT, D), jnp.float32)

    h = _pre_transform(x, W_pre, b_pre, W_pre2, b_pre2)

    part1, degp = _sc_agg_deg(h, src, dst, zacc)
    h1 = _combine("relu",
                  part1[0, :N], part1[1, :N],
                  degp[0, :N], degp[1, :N],
                  h, Wl1, bl1, Wr1)

    part2 = _sc_agg(h1, src, dst, zacc)
    out = _combine("l2norm",
                   part2[0, :N], part2[1, :N],
                   degp[0, :N], degp[1, :N],
                   h1, Wl2, bl2, Wr2)
    return out


# speedup vs baseline: 11.3359x; 1.3652x over previous
import functools

import jax
import jax.numpy as jnp
from jax import lax
from jax.experimental import pallas as pl
from jax.experimental.pallas import tpu as pltpu
from jax.experimental.pallas import tpu_sc as plsc

N = 10000
E = 320000
D = 128

NC = 2
NS = 16
NW = NC * NS
EW = E // NW
C = 125
K = EW // C
G = 4
KG = K // G
_STAGE = 3
NPAD = 10240
RPT = NPAD // NS
DW = 16


def _sc_agg_body(with_deg, h_hbm, src_hbm, dst_hbm, zacc_hbm, zdeg_hbm,
                 part_hbm, *rest):
    if with_deg:
        degp_hbm, acc_sp, deg_sp, sidx_g, didx_g, rows_a, rows_b, \
            ones1, sem_ga, sem_gb, sem_sa, sem_sb, sem_d = rest
    else:
        degp_hbm = deg_sp = ones1 = sem_d = None
        acc_sp, sidx_g, didx_g, rows_a, rows_b, \
            sem_ga, sem_gb, sem_sa, sem_sb = rest
    cid = lax.axis_index("c")
    sid = lax.axis_index("s")
    wid = sid * NC + cid

    if with_deg:
        pltpu.sync_copy(zdeg_hbm, deg_sp.at[pl.ds(sid * RPT, RPT)])
        ones16 = jnp.full((16,), 1.0, dtype=jnp.float32)

        def init_ones(i, _):
            ones1[pl.ds(i * 16, 16)] = ones16
            return _
        lax.fori_loop(0, C // 16, init_ones, None)
        ones1[pl.ds(C - 16, 16)] = ones16

    pltpu.sync_copy(zacc_hbm, acc_sp.at[pl.ds(sid * RPT, RPT)])

    plsc.subcore_barrier()

    def gather_start(k, rows, sem):
        pltpu.async_copy(h_hbm.at[sidx_g.at[k, 0]], rows, sem)

    def gather_wait(k, rows, sem):
        pltpu.make_async_copy(h_hbm.at[sidx_g.at[k, 0]], rows, sem).wait()

    def scat_start(k, rows, sem):
        pltpu.async_copy(rows, acc_sp.at[didx_g.at[k, 0]], sem, add=True)

    def scat_wait(rows, sem):
        pltpu.make_async_copy(rows, acc_sp.at[didx_g.at[0, 0]], sem).wait()

    P = KG // 2

    def group(g, _):
        base = wid * G + g
        pltpu.sync_copy(src_hbm.at[base], sidx_g)
        pltpu.sync_copy(dst_hbm.at[base], didx_g)

        if with_deg:
            def dfire(j, _):
                pltpu.async_copy(ones1, deg_sp.at[didx_g.at[j, 0]],
                                 sem_d, add=True)
                return _
            lax.fori_loop(0, KG, dfire, None)

        gather_start(0, rows_a, sem_ga)
        gather_start(1, rows_b, sem_gb)

        def pair(j, _):
            a = 2 * j
            gather_wait(a, rows_a, sem_ga)
            scat_start(a, rows_a, sem_sa)
            gather_wait(a + 1, rows_b, sem_gb)
            scat_start(a + 1, rows_b, sem_sb)
            scat_wait(rows_a, sem_sa)
            gather_start(a + 2, rows_a, sem_ga)
            scat_wait(rows_b, sem_sb)
            gather_start(a + 3, rows_b, sem_gb)
            return _
        lax.fori_loop(0, P - 1, pair, None)

        a = 2 * (P - 1)
        gather_wait(a, rows_a, sem_ga)
        scat_start(a, rows_a, sem_sa)
        gather_wait(a + 1, rows_b, sem_gb)
        scat_start(a + 1, rows_b, sem_sb)
        scat_wait(rows_a, sem_sa)
        scat_wait(rows_b, sem_sb)

        if with_deg:
            def ddrain(j, _):
                pltpu.make_async_copy(ones1, deg_sp.at[didx_g.at[0, 0]],
                                      sem_d).wait()
                return _
            lax.fori_loop(0, KG, ddrain, None)
        return _
    lax.fori_loop(0, G, group, None)

    plsc.subcore_barrier()

    pltpu.sync_copy(acc_sp.at[pl.ds(sid * RPT, RPT)],
                    part_hbm.at[cid, pl.ds(sid * RPT, RPT)])
    if with_deg:
        pltpu.sync_copy(deg_sp.at[pl.ds(sid * RPT, RPT)],
                        degp_hbm.at[pl.ds(cid * NPAD + sid * RPT, RPT)])


def _make_sc_agg(with_deg):
    mesh = plsc.VectorSubcoreMesh(core_axis_name="c", subcore_axis_name="s")
    out_type = [jax.ShapeDtypeStruct((NC, NPAD, D), jnp.float32)]
    scratch = [pltpu.VMEM_SHARED((NPAD, D), jnp.float32)]
    if with_deg:
        out_type.append(jax.ShapeDtypeStruct((NC * NPAD,), jnp.float32))
        scratch.append(pltpu.VMEM_SHARED((NPAD,), jnp.float32))
    scratch += [
        pltpu.VMEM((KG, 1, C), jnp.int32),
        pltpu.VMEM((KG, 1, C), jnp.int32),
        pltpu.VMEM((C, D), jnp.float32),
        pltpu.VMEM((C, D), jnp.float32),
    ]
    if with_deg:
        scratch.append(pltpu.VMEM((C,), jnp.float32))
    scratch += [
        pltpu.SemaphoreType.DMA,
        pltpu.SemaphoreType.DMA,
        pltpu.SemaphoreType.DMA,
        pltpu.SemaphoreType.DMA,
    ]
    if with_deg:
        scratch.append(pltpu.SemaphoreType.DMA)
    return pl.kernel(
        functools.partial(_sc_agg_body, with_deg),
        out_type=tuple(out_type) if with_deg else out_type[0],
        mesh=mesh,
        scratch_types=scratch)


_sc_agg_deg = _make_sc_agg(True)
_sc_agg = _make_sc_agg(False)


BN = 2000


def _pre_body(x_ref, wp_ref, bp_ref, wp2_ref, bp2_ref, o_ref):
    h = lax.dot_general(x_ref[...], wp_ref[...],
                        (((1,), (1,)), ((), ())),
                        preferred_element_type=jnp.float32)
    h = h + bp_ref[0:1, :]
    h = lax.dot_general(h, wp2_ref[...],
                        (((1,), (1,)), ((), ())),
                        preferred_element_type=jnp.float32)
    o_ref[...] = h + bp2_ref[0:1, :]


def _pre_transform(x, Wp, bp, Wp2, bp2):
    grid = (N // BN,)
    row_spec = pl.BlockSpec((BN, D), lambda i: (i, 0))
    full = pl.BlockSpec((D, D), lambda i: (0, 0))
    bias = pl.BlockSpec((8, D), lambda i: (0, 0))
    return pl.pallas_call(
        _pre_body,
        grid=grid,
        in_specs=[row_spec, full, bias, full, bias],
        out_specs=row_spec,
        out_shape=jax.ShapeDtypeStruct((N, D), jnp.float32),
    )(x, Wp, jnp.broadcast_to(bp, (8, D)), Wp2, jnp.broadcast_to(bp2, (8, D)))


def _combine_body(act, p0_ref, p1_ref, d0_ref, d1_ref, h_ref,
                  wl_ref, bl_ref, wr_ref, o_ref):
    agg = p0_ref[...] + p1_ref[...]
    deg = d0_ref[...] + d1_ref[...]
    mean = agg / jnp.maximum(deg, 1.0)
    out = lax.dot_general(mean, wl_ref[...], (((1,), (1,)), ((), ())),
                          preferred_element_type=jnp.float32)
    out = out + bl_ref[0:1, :]
    out = out + lax.dot_general(h_ref[...], wr_ref[...],
                                (((1,), (1,)), ((), ())),
                                preferred_element_type=jnp.float32)
    if act == "relu":
        out = jnp.maximum(out, 0.0)
    elif act == "l2norm":
        nrm = jnp.sqrt(jnp.sum(out * out, axis=1, keepdims=True))
        out = out / jnp.maximum(nrm, 1e-12)
    o_ref[...] = out


def _combine(act, p0, p1, d0, d1, h, Wl, bl, Wr):
    grid = (N // BN,)
    row_spec = pl.BlockSpec((BN, D), lambda i: (i, 0))
    deg_spec = pl.BlockSpec((BN, 1), lambda i: (i, 0))
    full = pl.BlockSpec((D, D), lambda i: (0, 0))
    bias = pl.BlockSpec((8, D), lambda i: (0, 0))
    return pl.pallas_call(
        functools.partial(_combine_body, act),
        grid=grid,
        in_specs=[row_spec, row_spec, deg_spec, deg_spec, row_spec,
                  full, bias, full],
        out_specs=row_spec,
        out_shape=jax.ShapeDtypeStruct((N, D), jnp.float32),
    )(p0, p1, d0, d1, h, Wl, jnp.broadcast_to(bl, (8, D)), Wr)


def kernel(x, edge_index, W_pre, b_pre, W_pre2, b_pre2,
           Wl1, bl1, Wr1, Wl2, bl2, Wr2):
    src = edge_index[0].reshape(NW * G, KG, 1, C)
    dst = edge_index[1].reshape(NW * G, KG, 1, C)
    zacc = jnp.zeros((RPT, D), jnp.float32)
    zdeg = jnp.zeros((RPT,), jnp.float32)

    h = _pre_transform(x, W_pre, b_pre, W_pre2, b_pre2)

    part1, degp = _sc_agg_deg(h, src, dst, zacc, zdeg)
    d0 = degp[:N].reshape(N, 1)
    d1 = degp[NPAD:NPAD + N].reshape(N, 1)
    h1 = _combine("relu",
                  part1[0, :N], part1[1, :N], d0, d1,
                  h, Wl1, bl1, Wr1)

    part2 = _sc_agg(h1, src, dst, zacc, zdeg)
    out = _combine("l2norm",
                   part2[0, :N], part2[1, :N], d0, d1,
                   h1, Wl2, bl2, Wr2)
    return out
